# baseline (speedup 1.0000x reference)
import functools

import jax
import jax.numpy as jnp
from jax import lax
from jax.experimental import pallas as pl
from jax.experimental.pallas import tpu as pltpu
from jax.experimental.pallas import tpu_sc as plsc

EMBED_DIM = 64
BATCH = 16384
IDX_CHUNK = 128


@functools.cache
def _build(num_users: int, num_items: int):
    info = plsc.get_sparse_core_info()
    nc, ns, lanes = info.num_cores, info.num_subcores, info.num_lanes
    nw = nc * ns
    b_per_w = BATCH // nw
    n_chunks = b_per_w // IDX_CHUNK
    n_groups = b_per_w // lanes

    mesh = plsc.VectorSubcoreMesh(core_axis_name="c", subcore_axis_name="s")

    @functools.partial(
        pl.kernel,
        out_type=jax.ShapeDtypeStruct((BATCH,), jnp.float32),
        mesh=mesh,
        compiler_params=pltpu.CompilerParams(needs_layout_passes=False,
                                             use_tc_tiling_on_sc=False),
        scratch_types=[
            pltpu.VMEM((n_chunks, IDX_CHUNK), jnp.int32),
            pltpu.VMEM((n_chunks, IDX_CHUNK), jnp.int32),
            pltpu.VMEM((b_per_w, EMBED_DIM), jnp.float32),
            pltpu.VMEM((b_per_w, EMBED_DIM), jnp.float32),
            pltpu.VMEM((b_per_w,), jnp.float32),
            pltpu.SemaphoreType.DMA,
        ],
    )
    def sc_kernel(user_hbm, item_hbm, utab_hbm, itab_hbm, out_hbm,
                  uidx, iidx, urows, irows, outv, sem):
        wid = lax.axis_index("s") * nc + lax.axis_index("c")
        base = wid * b_per_w

        for j in range(n_chunks):
            pltpu.sync_copy(user_hbm.at[pl.ds(base + j * IDX_CHUNK, IDX_CHUNK)],
                            uidx.at[j])
            pltpu.sync_copy(item_hbm.at[pl.ds(base + j * IDX_CHUNK, IDX_CHUNK)],
                            iidx.at[j])

        copies = []
        for j in range(n_chunks):
            dst = pl.ds(j * IDX_CHUNK, IDX_CHUNK)
            copies.append(pltpu.async_copy(utab_hbm.at[uidx.at[j]],
                                           urows.at[dst], sem))
            copies.append(pltpu.async_copy(itab_hbm.at[iidx.at[j]],
                                           irows.at[dst], sem))
        for c in copies:
            c.wait()

        lane = jax.lax.iota(jnp.int32, lanes)

        def group_body(g, _):
            row = g * lanes + lane
            acc = jnp.zeros((lanes,), jnp.float32)
            for d in range(EMBED_DIM):
                col = jnp.full((lanes,), d, jnp.int32)
                u = plsc.load_gather(urows, [row, col])
                v = plsc.load_gather(irows, [row, col])
                acc = acc + u * v
            outv[pl.ds(g * lanes, lanes)] = acc
            return 0

        lax.fori_loop(0, n_groups, group_body, 0)

        pltpu.sync_copy(outv, out_hbm.at[pl.ds(base, b_per_w)])

    return sc_kernel


def kernel(user, item, user_table, item_table):
    fn = _build(user_table.shape[0], item_table.shape[0])
    return fn(user.astype(jnp.int32), item.astype(jnp.int32),
              user_table, item_table)
